# Initial kernel scaffold; baseline (speedup 1.0000x reference)
import jax
import jax.numpy as jnp
from jax.experimental import pallas as pl


def kernel(x):
    raise NotImplementedError("write your pallas kernel here")


# speedup vs baseline: 1.6541x; 1.6541x over previous
import functools

import jax
import jax.numpy as jnp
from jax.experimental import pallas as pl
from jax.experimental.pallas import tpu as pltpu

L = 8192
F = L // 2 + 1
K = 4

BF = 512
NFB = 9
KB = 1024
NKB = L // KB
LC = 1024
NL0 = L // LC

_TWO_PI_OVER_L = 2.0 * 3.14159265358979323846 / L


def _dft_topk_kernel(x_ref, val_ref, f_ref, re_ref, im_ref, racc, iacc):
    fb = pl.program_id(1)
    kb = pl.program_id(2)
    f0 = fb * BF
    k0 = kb * KB

    frow = jax.lax.broadcasted_iota(jnp.int32, (BF, KB), 0) + f0
    lcol = jax.lax.broadcasted_iota(jnp.int32, (BF, KB), 1) + k0
    m = (frow * lcol) & (L - 1)
    ang = m.astype(jnp.float32) * _TWO_PI_OVER_L
    cosb = jnp.cos(ang)
    sinb = -jnp.sin(ang)

    xb = x_ref[0]
    pr = jnp.dot(cosb, xb, preferred_element_type=jnp.float32,
                 precision=jax.lax.Precision.HIGHEST)
    pi = jnp.dot(sinb, xb, preferred_element_type=jnp.float32,
                 precision=jax.lax.Precision.HIGHEST)

    @pl.when(kb == 0)
    def _():
        racc[...] = pr
        iacc[...] = pi

    @pl.when(kb > 0)
    def _():
        racc[...] += pr
        iacc[...] += pi

    @pl.when(kb == NKB - 1)
    def _():
        re = racc[...]
        im = iacc[...]
        amp2 = re * re + im * im
        d = re.shape[1]
        rowid = jax.lax.broadcasted_iota(jnp.int32, (BF, d), 0)
        valid = (rowid + f0) < F
        work = jnp.where(valid, amp2, -1.0)
        for r in range(K):
            mx = jnp.max(work, axis=0)
            cand = jnp.where(work == mx[None, :], rowid, jnp.int32(2**30))
            idx = jnp.min(cand, axis=0)
            oh = rowid == idx[None, :]
            val_ref[0, 0, r, :] = mx
            f_ref[0, 0, r, :] = (idx + f0).astype(jnp.float32)
            re_ref[0, 0, r, :] = jnp.sum(jnp.where(oh, re, 0.0), axis=0)
            im_ref[0, 0, r, :] = jnp.sum(jnp.where(oh, im, 0.0), axis=0)
            work = jnp.where(oh, -1.0, work)


def _recon_kernel(val_ref, f_ref, re_ref, im_ref, out_ref,
                  fsel, asel, bsel, cosA, sinA):
    l0 = pl.program_id(1)
    d = out_ref.shape[2]

    @pl.when(l0 == 0)
    def _():
        vals = val_ref[0].reshape(NFB * K, d)
        fs = f_ref[0].reshape(NFB * K, d)
        res = re_ref[0].reshape(NFB * K, d)
        ims = im_ref[0].reshape(NFB * K, d)
        rowid = jax.lax.broadcasted_iota(jnp.int32, (NFB * K, d), 0)
        work = vals
        for r in range(K):
            mx = jnp.max(work, axis=0)
            cand = jnp.where(work == mx[None, :], rowid, jnp.int32(2**30))
            idx = jnp.min(cand, axis=0)
            oh = rowid == idx[None, :]
            fr = jnp.sum(jnp.where(oh, fs, 0.0), axis=0)
            rr = jnp.sum(jnp.where(oh, res, 0.0), axis=0)
            ir = jnp.sum(jnp.where(oh, ims, 0.0), axis=0)
            w = jnp.where((fr == 0.0) | (fr == float(L // 2)), 1.0, 2.0) / L
            fsel[r, :] = fr.astype(jnp.int32)
            asel[r, :] = w * rr
            bsel[r, :] = -w * ir
            work = jnp.where(oh, -1.0, work)
        for r in range(K):
            fi = fsel[r, :][None, :]
            dl = jax.lax.broadcasted_iota(jnp.int32, (LC, d), 0)
            mm = (dl * fi) & (L - 1)
            a = mm.astype(jnp.float32) * _TWO_PI_OVER_L
            cosA[r] = jnp.cos(a)
            sinA[r] = jnp.sin(a)

    acc = jnp.zeros((LC, d), jnp.float32)
    for r in range(K):
        fi = fsel[r, :]
        ph = (fi * (LC * l0)) & (L - 1)
        a0 = ph.astype(jnp.float32) * _TWO_PI_OVER_L
        cp = jnp.cos(a0)[None, :]
        sp = jnp.sin(a0)[None, :]
        A = asel[r, :][None, :]
        Bc = bsel[r, :][None, :]
        U = A * cp + Bc * sp
        V = Bc * cp - A * sp
        acc = acc + U * cosA[r] + V * sinA[r]
    out_ref[0] = acc


def kernel(x):
    b, l, d = x.shape
    assert l == L

    cand_shape = jax.ShapeDtypeStruct((b, NFB, K, d), jnp.float32)
    val, fsel, re, im = pl.pallas_call(
        _dft_topk_kernel,
        grid=(b, NFB, NKB),
        in_specs=[pl.BlockSpec((1, KB, d), lambda bi, fb, kb: (bi, kb, 0))],
        out_specs=[pl.BlockSpec((1, 1, K, d), lambda bi, fb, kb: (bi, fb, 0, 0))] * 4,
        out_shape=[cand_shape] * 4,
        scratch_shapes=[pltpu.VMEM((BF, d), jnp.float32),
                        pltpu.VMEM((BF, d), jnp.float32)],
        compiler_params=pltpu.CompilerParams(
            dimension_semantics=("arbitrary", "arbitrary", "arbitrary"),
            vmem_limit_bytes=100 * 1024 * 1024,
        ),
    )(x)

    cand_spec = pl.BlockSpec((1, NFB, K, d), lambda bi, l0: (bi, 0, 0, 0))
    xhat = pl.pallas_call(
        _recon_kernel,
        grid=(b, NL0),
        in_specs=[cand_spec] * 4,
        out_specs=pl.BlockSpec((1, LC, d), lambda bi, l0: (bi, l0, 0)),
        out_shape=jax.ShapeDtypeStruct((b, l, d), jnp.float32),
        scratch_shapes=[pltpu.VMEM((K, d), jnp.int32),
                        pltpu.VMEM((K, d), jnp.float32),
                        pltpu.VMEM((K, d), jnp.float32),
                        pltpu.VMEM((K, LC, d), jnp.float32),
                        pltpu.VMEM((K, LC, d), jnp.float32)],
        compiler_params=pltpu.CompilerParams(
            dimension_semantics=("arbitrary", "arbitrary"),
        ),
    )(val, fsel, re, im)
    return xhat


# speedup vs baseline: 4.5901x; 2.7751x over previous
import functools

import jax
import jax.numpy as jnp
from jax.experimental import pallas as pl
from jax.experimental.pallas import tpu as pltpu

L = 8192
F = L // 2 + 1
K = 4

NCLS = 4
NJ = L // NCLS
FH = NJ // 2
BF = 352
NFB = 3
KBJ = 512
NKB = NJ // KBJ
LC = 1024
NL0 = L // LC

_PI = 3.14159265358979323846


def _dft_topk_kernel(x_ref, val_ref, f_ref, re_ref, im_ref, acc):
    fb = pl.program_id(1)
    kb = pl.program_id(2)
    f0 = fb * BF
    j0 = kb * KBJ

    frow = jax.lax.broadcasted_iota(jnp.int32, (BF, KBJ), 0) + f0
    jcol = jax.lax.broadcasted_iota(jnp.int32, (BF, KBJ), 1) + j0
    m = (frow * jcol) & (NJ - 1)
    ang = m.astype(jnp.float32) * (2.0 * _PI / NJ)
    bas = jnp.concatenate([jnp.cos(ang), -jnp.sin(ang)], axis=0)

    for c in range(NCLS):
        xc = x_ref[0, :, c, :]
        p = jnp.dot(bas, xc, preferred_element_type=jnp.float32,
                    precision=jax.lax.Precision.HIGHEST)

        @pl.when(kb == 0)
        def _():
            acc[c] = p

        @pl.when(kb > 0)
        def _():
            acc[c] += p

    @pl.when(kb == NKB - 1)
    def _():
        d = x_ref.shape[3]
        fcol = jax.lax.broadcasted_iota(jnp.int32, (BF, 1), 0) + f0
        fsubs = [fcol, 2 * FH - fcol, 2 * FH + fcol, 4 * FH - fcol]
        conjs = [False, True, False, True]
        res, ims, fls = [], [], []
        for s in range(4):
            fm = fsubs[s]
            sgn = -1.0 if conjs[s] else 1.0
            rs = acc[0, 0:BF, :]
            is_ = acc[0, BF:2 * BF, :] * sgn
            for c in range(1, NCLS):
                angt = ((fm * c) & (L - 1)).astype(jnp.float32) * (2.0 * _PI / L)
                tre = jnp.cos(angt)
                tim = -jnp.sin(angt)
                gre = acc[c, 0:BF, :]
                gim = acc[c, BF:2 * BF, :] * sgn
                rs = rs + tre * gre - tim * gim
                is_ = is_ + tre * gim + tim * gre
            res.append(rs)
            ims.append(is_)
            fls.append(fm)
        re_all = jnp.concatenate(res, axis=0)
        im_all = jnp.concatenate(ims, axis=0)
        f_all = jnp.concatenate(fls, axis=0)

        inrange = (fcol <= FH)
        valids = [inrange,
                  inrange & (fcol != FH),
                  inrange & (fcol != 0),
                  inrange & (fcol != FH)]
        valid = jnp.concatenate(valids, axis=0)

        amp2 = re_all * re_all + im_all * im_all
        work = jnp.where(valid, amp2, -1.0)
        rowid = jax.lax.broadcasted_iota(jnp.int32, (4 * BF, d), 0)
        fb_all = jnp.broadcast_to(f_all, (4 * BF, d)).astype(jnp.float32)
        for r in range(K):
            mx = jnp.max(work, axis=0)
            cand = jnp.where(work == mx[None, :], rowid, jnp.int32(2**30))
            idx = jnp.min(cand, axis=0)
            oh = rowid == idx[None, :]
            val_ref[0, 0, r, :] = mx
            f_ref[0, 0, r, :] = jnp.sum(jnp.where(oh, fb_all, 0.0), axis=0)
            re_ref[0, 0, r, :] = jnp.sum(jnp.where(oh, re_all, 0.0), axis=0)
            im_ref[0, 0, r, :] = jnp.sum(jnp.where(oh, im_all, 0.0), axis=0)
            work = jnp.where(oh, -1.0, work)


def _recon_kernel(val_ref, f_ref, re_ref, im_ref, out_ref,
                  fsel, asel, bsel, cosA, sinA):
    l0 = pl.program_id(1)
    d = out_ref.shape[2]

    @pl.when(l0 == 0)
    def _():
        vals = val_ref[0].reshape(NFB * K, d)
        fs = f_ref[0].reshape(NFB * K, d)
        res = re_ref[0].reshape(NFB * K, d)
        ims = im_ref[0].reshape(NFB * K, d)
        rowid = jax.lax.broadcasted_iota(jnp.int32, (NFB * K, d), 0)
        work = vals
        for r in range(K):
            mx = jnp.max(work, axis=0)
            cand = jnp.where(work == mx[None, :], rowid, jnp.int32(2**30))
            idx = jnp.min(cand, axis=0)
            oh = rowid == idx[None, :]
            fr = jnp.sum(jnp.where(oh, fs, 0.0), axis=0)
            rr = jnp.sum(jnp.where(oh, res, 0.0), axis=0)
            ir = jnp.sum(jnp.where(oh, ims, 0.0), axis=0)
            w = jnp.where((fr == 0.0) | (fr == float(L // 2)), 1.0, 2.0) / L
            fsel[r, :] = fr.astype(jnp.int32)
            asel[r, :] = w * rr
            bsel[r, :] = -w * ir
            work = jnp.where(oh, -1.0, work)
        for r in range(K):
            fi = fsel[r, :][None, :]
            dl = jax.lax.broadcasted_iota(jnp.int32, (LC, d), 0)
            mm = (dl * fi) & (L - 1)
            a = mm.astype(jnp.float32) * (2.0 * _PI / L)
            cosA[r] = jnp.cos(a)
            sinA[r] = jnp.sin(a)

    acc = jnp.zeros((LC, d), jnp.float32)
    for r in range(K):
        fi = fsel[r, :]
        ph = (fi * (LC * l0)) & (L - 1)
        a0 = ph.astype(jnp.float32) * (2.0 * _PI / L)
        cp = jnp.cos(a0)[None, :]
        sp = jnp.sin(a0)[None, :]
        A = asel[r, :][None, :]
        Bc = bsel[r, :][None, :]
        U = A * cp + Bc * sp
        V = Bc * cp - A * sp
        acc = acc + U * cosA[r] + V * sinA[r]
    out_ref[0] = acc


def kernel(x):
    b, l, d = x.shape
    assert l == L
    xr = x.reshape(b, NJ, NCLS, d)

    cand_shape = jax.ShapeDtypeStruct((b, NFB, K, d), jnp.float32)
    val, fsel, re, im = pl.pallas_call(
        _dft_topk_kernel,
        grid=(b, NFB, NKB),
        in_specs=[pl.BlockSpec((1, KBJ, NCLS, d),
                               lambda bi, fb, kb: (bi, kb, 0, 0))],
        out_specs=[pl.BlockSpec((1, 1, K, d),
                                lambda bi, fb, kb: (bi, fb, 0, 0))] * 4,
        out_shape=[cand_shape] * 4,
        scratch_shapes=[pltpu.VMEM((NCLS, 2 * BF, d), jnp.float32)],
        compiler_params=pltpu.CompilerParams(
            dimension_semantics=("arbitrary", "arbitrary", "arbitrary"),
            vmem_limit_bytes=100 * 1024 * 1024,
        ),
    )(xr)

    cand_spec = pl.BlockSpec((1, NFB, K, d), lambda bi, l0: (bi, 0, 0, 0))
    xhat = pl.pallas_call(
        _recon_kernel,
        grid=(b, NL0),
        in_specs=[cand_spec] * 4,
        out_specs=pl.BlockSpec((1, LC, d), lambda bi, l0: (bi, l0, 0)),
        out_shape=jax.ShapeDtypeStruct((b, l, d), jnp.float32),
        scratch_shapes=[pltpu.VMEM((K, d), jnp.int32),
                        pltpu.VMEM((K, d), jnp.float32),
                        pltpu.VMEM((K, d), jnp.float32),
                        pltpu.VMEM((K, LC, d), jnp.float32),
                        pltpu.VMEM((K, LC, d), jnp.float32)],
        compiler_params=pltpu.CompilerParams(
            dimension_semantics=("arbitrary", "arbitrary"),
            vmem_limit_bytes=100 * 1024 * 1024,
        ),
    )(val, fsel, re, im)
    return xhat


# speedup vs baseline: 5.4895x; 1.1959x over previous
import functools

import jax
import jax.numpy as jnp
from jax.experimental import pallas as pl
from jax.experimental.pallas import tpu as pltpu

L = 8192
F = L // 2 + 1
K = 4

NCLS = 8
NJ = L // NCLS
FH = NJ // 2
BF = 264
NFB = 2
NCAND = 8 * K
KBJ = 256
NKB = NJ // KBJ
LC = 1024
NL0 = L // LC

_PI = 3.14159265358979323846


def _dft_topk_kernel(x_ref, val_ref, f_ref, re_ref, im_ref, acc):
    fb = pl.program_id(1)
    kb = pl.program_id(2)
    f0 = fb * BF
    j0 = kb * KBJ

    frow = jax.lax.broadcasted_iota(jnp.int32, (BF, KBJ), 0) + f0
    jcol = jax.lax.broadcasted_iota(jnp.int32, (BF, KBJ), 1) + j0
    m = (frow * jcol) & (NJ - 1)
    ang = m.astype(jnp.float32) * (2.0 * _PI / NJ)
    bas = jnp.concatenate([jnp.cos(ang), -jnp.sin(ang)], axis=0)

    for c in range(NCLS):
        xc = x_ref[0, :, c, :]
        p = jnp.dot(bas, xc, preferred_element_type=jnp.float32,
                    precision=jax.lax.Precision.HIGHEST)

        @pl.when(kb == 0)
        def _():
            acc[c] = p

        @pl.when(kb > 0)
        def _():
            acc[c] += p

    @pl.when(kb == NKB - 1)
    def _():
        d = x_ref.shape[3]
        fcol = jax.lax.broadcasted_iota(jnp.int32, (BF, 1), 0) + f0
        rowid = jax.lax.broadcasted_iota(jnp.int32, (BF, d), 0)
        inrange = fcol <= FH
        sub = 0
        for q in range(5):
            for sgnform in (0, 1):
                if sgnform == 0 and q == 4:
                    continue
                if sgnform == 1 and q == 0:
                    continue
                conj = sgnform == 1
                fm = NJ * q + fcol if not conj else NJ * q - fcol
                sgn = -1.0 if conj else 1.0
                rs = acc[0, 0:BF, :]
                is_ = acc[0, BF:2 * BF, :] * sgn
                for c in range(1, NCLS):
                    angt = (((fm * c) & (L - 1)).astype(jnp.float32)
                            * (2.0 * _PI / L))
                    tre = jnp.cos(angt)
                    tim = -jnp.sin(angt)
                    gre = acc[c, 0:BF, :]
                    gim = acc[c, BF:2 * BF, :] * sgn
                    rs = rs + tre * gre - tim * gim
                    is_ = is_ + tre * gim + tim * gre
                valid = inrange
                if conj:
                    if q < 4:
                        valid = valid & (fcol != 0)
                    valid = valid & (fcol != FH)
                amp2 = rs * rs + is_ * is_
                work = jnp.where(valid, amp2, -1.0)
                fmf = jnp.broadcast_to(fm, (BF, d)).astype(jnp.float32)
                for r in range(K):
                    mx = jnp.max(work, axis=0)
                    cand = jnp.where(work == mx[None, :], rowid,
                                     jnp.int32(2**30))
                    idx = jnp.min(cand, axis=0)
                    oh = rowid == idx[None, :]
                    slot = sub * K + r
                    val_ref[0, 0, slot, :] = mx
                    f_ref[0, 0, slot, :] = jnp.sum(
                        jnp.where(oh, fmf, 0.0), axis=0)
                    re_ref[0, 0, slot, :] = jnp.sum(
                        jnp.where(oh, rs, 0.0), axis=0)
                    im_ref[0, 0, slot, :] = jnp.sum(
                        jnp.where(oh, is_, 0.0), axis=0)
                    work = jnp.where(oh, -1.0, work)
                sub += 1


def _recon_kernel(val_ref, f_ref, re_ref, im_ref, out_ref,
                  fsel, asel, bsel, cosA, sinA):
    l0 = pl.program_id(1)
    d = out_ref.shape[2]

    @pl.when(l0 == 0)
    def _():
        nc = NFB * NCAND
        vals = val_ref[0].reshape(nc, d)
        fs = f_ref[0].reshape(nc, d)
        res = re_ref[0].reshape(nc, d)
        ims = im_ref[0].reshape(nc, d)
        rowid = jax.lax.broadcasted_iota(jnp.int32, (nc, d), 0)
        work = vals
        for r in range(K):
            mx = jnp.max(work, axis=0)
            cand = jnp.where(work == mx[None, :], rowid, jnp.int32(2**30))
            idx = jnp.min(cand, axis=0)
            oh = rowid == idx[None, :]
            fr = jnp.sum(jnp.where(oh, fs, 0.0), axis=0)
            rr = jnp.sum(jnp.where(oh, res, 0.0), axis=0)
            ir = jnp.sum(jnp.where(oh, ims, 0.0), axis=0)
            w = jnp.where((fr == 0.0) | (fr == float(L // 2)), 1.0, 2.0) / L
            fsel[r, :] = fr.astype(jnp.int32)
            asel[r, :] = w * rr
            bsel[r, :] = -w * ir
            work = jnp.where(oh, -1.0, work)
        for r in range(K):
            fi = fsel[r, :][None, :]
            dl = jax.lax.broadcasted_iota(jnp.int32, (LC, d), 0)
            mm = (dl * fi) & (L - 1)
            a = mm.astype(jnp.float32) * (2.0 * _PI / L)
            cosA[r] = jnp.cos(a)
            sinA[r] = jnp.sin(a)

    acc = jnp.zeros((LC, d), jnp.float32)
    for r in range(K):
        fi = fsel[r, :]
        ph = (fi * (LC * l0)) & (L - 1)
        a0 = ph.astype(jnp.float32) * (2.0 * _PI / L)
        cp = jnp.cos(a0)[None, :]
        sp = jnp.sin(a0)[None, :]
        A = asel[r, :][None, :]
        Bc = bsel[r, :][None, :]
        U = A * cp + Bc * sp
        V = Bc * cp - A * sp
        acc = acc + U * cosA[r] + V * sinA[r]
    out_ref[0] = acc


def kernel(x):
    b, l, d = x.shape
    assert l == L
    xr = x.reshape(b, NJ, NCLS, d)

    cand_shape = jax.ShapeDtypeStruct((b, NFB, NCAND, d), jnp.float32)
    val, fsel, re, im = pl.pallas_call(
        _dft_topk_kernel,
        grid=(b, NFB, NKB),
        in_specs=[pl.BlockSpec((1, KBJ, NCLS, d),
                               lambda bi, fb, kb: (bi, kb, 0, 0))],
        out_specs=[pl.BlockSpec((1, 1, NCAND, d),
                                lambda bi, fb, kb: (bi, fb, 0, 0))] * 4,
        out_shape=[cand_shape] * 4,
        scratch_shapes=[pltpu.VMEM((NCLS, 2 * BF, d), jnp.float32)],
        compiler_params=pltpu.CompilerParams(
            dimension_semantics=("arbitrary", "arbitrary", "arbitrary"),
            vmem_limit_bytes=60 * 1024 * 1024,
        ),
    )(xr)

    cand_spec = pl.BlockSpec((1, NFB, NCAND, d), lambda bi, l0: (bi, 0, 0, 0))
    xhat = pl.pallas_call(
        _recon_kernel,
        grid=(b, NL0),
        in_specs=[cand_spec] * 4,
        out_specs=pl.BlockSpec((1, LC, d), lambda bi, l0: (bi, l0, 0)),
        out_shape=jax.ShapeDtypeStruct((b, l, d), jnp.float32),
        scratch_shapes=[pltpu.VMEM((K, d), jnp.int32),
                        pltpu.VMEM((K, d), jnp.float32),
                        pltpu.VMEM((K, d), jnp.float32),
                        pltpu.VMEM((K, LC, d), jnp.float32),
                        pltpu.VMEM((K, LC, d), jnp.float32)],
        compiler_params=pltpu.CompilerParams(
            dimension_semantics=("arbitrary", "arbitrary"),
            vmem_limit_bytes=60 * 1024 * 1024,
        ),
    )(val, fsel, re, im)
    return xhat
